# baseline (device time: 62372 ns/iter reference)
import jax
import jax.numpy as jnp
from jax import lax
from jax.experimental import pallas as pl
from jax.experimental.pallas import tpu as pltpu

B, H, D, BS = 16, 16, 64, 16
NB = 128
NPAGES_LOCAL = 128
NKEYS = NPAGES_LOCAL * BS
HD = H * D
MCOL = HD
LCOL = HD + H
CW = HD + 2 * H

NEG = -1e30


def _body(q_ref, k_ref, v_ref, bt_ref, lens_ref, out_ref,
          comm_out, comm_in, send_sem, recv_sem):
    my_x = lax.axis_index("x")
    peer = (1 - my_x, lax.axis_index("y"), lax.axis_index("z"))

    barrier = pltpu.get_barrier_semaphore()
    pl.semaphore_signal(barrier, inc=1, device_id=peer,
                        device_id_type=pl.DeviceIdType.MESH)
    pl.semaphore_wait(barrier, 1)

    bt3 = bt_ref[:][:, :, None]
    pages = lax.broadcasted_iota(jnp.int32, (B, NB, NPAGES_LOCAL), 2) \
        + my_x * NPAGES_LOCAL
    slots = lax.broadcasted_iota(jnp.int32, (B, NB, NPAGES_LOCAL), 1)
    lens3 = lens_ref[:][:, :, None]
    hit = (bt3 == pages) & (slots < lens3)
    cnt = jnp.sum(hit.astype(jnp.float32), axis=1)

    pr = lax.broadcasted_iota(jnp.int32, (NPAGES_LOCAL, NKEYS), 0)
    kc = lax.broadcasted_iota(jnp.int32, (NPAGES_LOCAL, NKEYS), 1)
    expand = (pr == kc // BS).astype(jnp.float32)
    w = lax.dot_general(cnt, expand, (((1,), (0,)), ((), ())),
                        preferred_element_type=jnp.float32)
    neg = jnp.where(w > 0.0, 0.0, NEG)

    scale = D ** -0.5
    for h in range(H):
        q = q_ref[:, h * D:(h + 1) * D]
        k = k_ref[:, h * D:(h + 1) * D]
        s = lax.dot_general(q, k, (((1,), (1,)), ((), ())),
                            preferred_element_type=jnp.float32) * scale
        s = s + neg
        m = jnp.max(s, axis=1, keepdims=True)
        p = jnp.exp(s - m) * w
        l = jnp.sum(p, axis=1, keepdims=True)
        v = v_ref[:, h * D:(h + 1) * D]
        o = lax.dot_general(p, v, (((1,), (0,)), ((), ())),
                            preferred_element_type=jnp.float32)
        comm_out[:, h * D:(h + 1) * D] = o
        comm_out[:, MCOL + h:MCOL + h + 1] = m
        comm_out[:, LCOL + h:LCOL + h + 1] = l

    rdma = pltpu.make_async_remote_copy(
        src_ref=comm_out, dst_ref=comm_in,
        send_sem=send_sem, recv_sem=recv_sem,
        device_id=peer, device_id_type=pl.DeviceIdType.MESH,
    )
    rdma.start()
    rdma.wait()

    m_l = comm_out[:, MCOL:MCOL + H]
    l_l = comm_out[:, LCOL:LCOL + H]
    m_r = comm_in[:, MCOL:MCOL + H]
    l_r = comm_in[:, LCOL:LCOL + H]
    m_c = jnp.maximum(m_l, m_r)
    a_l = jnp.exp(m_l - m_c)
    a_r = jnp.exp(m_r - m_c)
    l_c = a_l * l_l + a_r * l_r

    hr = lax.broadcasted_iota(jnp.int32, (H, HD), 0)
    hc = lax.broadcasted_iota(jnp.int32, (H, HD), 1)
    e2 = (hr == hc // D).astype(jnp.float32)

    def exp_cols(a):
        return lax.dot_general(a, e2, (((1,), (0,)), ((), ())),
                               preferred_element_type=jnp.float32)

    out_ref[:, :] = (exp_cols(a_l) * comm_out[:, :HD]
                     + exp_cols(a_r) * comm_in[:, :HD]) / exp_cols(l_c)


def kernel(Q, K, V, bt, lens):
    q2 = Q.reshape(B, HD)
    k2 = K.reshape(NKEYS, HD)
    v2 = V.reshape(NKEYS, HD)
    lens2 = lens.reshape(B, 1)
    out = pl.pallas_call(
        _body,
        out_shape=jax.ShapeDtypeStruct((B, HD), jnp.float32),
        in_specs=[pl.BlockSpec(memory_space=pltpu.VMEM)] * 5,
        out_specs=pl.BlockSpec(memory_space=pltpu.VMEM),
        scratch_shapes=[
            pltpu.VMEM((B, CW), jnp.float32),
            pltpu.VMEM((B, CW), jnp.float32),
            pltpu.SemaphoreType.DMA,
            pltpu.SemaphoreType.DMA,
        ],
        compiler_params=pltpu.CompilerParams(collective_id=0),
    )(q2, k2, v2, bt, lens2)
    return out.reshape(B, 1, H, D)


# device time: 57931 ns/iter; 1.0767x vs baseline; 1.0767x over previous
import jax
import jax.numpy as jnp
from jax import lax
from jax.experimental import pallas as pl
from jax.experimental.pallas import tpu as pltpu

B, H, D, BS = 16, 16, 64, 16
NB = 128
NPAGES_LOCAL = 128
NKEYS = NPAGES_LOCAL * BS
HD = H * D
MCOL = HD
LCOL = HD + H
CW = HD + 2 * H

NEG = -1e30


def _body(q_ref, k_ref, v_ref, bt_ref, lens_ref, out_ref,
          comm_out, comm_in, send_sem, recv_sem):
    my_x = lax.axis_index("x")
    peer = (1 - my_x, lax.axis_index("y"), lax.axis_index("z"))

    barrier = pltpu.get_barrier_semaphore()
    pl.semaphore_signal(barrier, inc=1, device_id=peer,
                        device_id_type=pl.DeviceIdType.MESH)
    pl.semaphore_wait(barrier, 1)

    bt3 = bt_ref[:][:, :, None]
    pages = lax.broadcasted_iota(jnp.int32, (B, NB, NPAGES_LOCAL), 2) \
        + my_x * NPAGES_LOCAL
    slots = lax.broadcasted_iota(jnp.int32, (B, NB, NPAGES_LOCAL), 1)
    lens3 = lens_ref[:][:, :, None]
    hit = (bt3 == pages) & (slots < lens3)
    cnt = jnp.sum(hit.astype(jnp.float32), axis=1)

    pr = lax.broadcasted_iota(jnp.int32, (NPAGES_LOCAL, NKEYS), 0)
    kc = lax.broadcasted_iota(jnp.int32, (NPAGES_LOCAL, NKEYS), 1)
    expand = (pr == kc // BS).astype(jnp.float32)
    w = lax.dot_general(cnt, expand, (((1,), (0,)), ((), ())),
                        preferred_element_type=jnp.float32)

    scale = D ** -0.5
    q_t = jnp.broadcast_to(q_ref[:][None], (H, B, HD)).reshape(H * B, HD)
    rowh = lax.broadcasted_iota(jnp.int32, (H * B, HD), 0) // B
    colh = lax.broadcasted_iota(jnp.int32, (H * B, HD), 1) // D
    qbig = jnp.where(rowh == colh, q_t, 0.0).astype(jnp.bfloat16)
    k16 = k_ref[:].astype(jnp.bfloat16)
    w_t = jnp.broadcast_to(w[None], (H, B, NKEYS)).reshape(H * B, NKEYS)
    neg_t = jnp.where(w_t > 0.0, 0.0, NEG)

    s = lax.dot_general(qbig, k16, (((1,), (1,)), ((), ())),
                        preferred_element_type=jnp.float32) * scale
    s = s + neg_t
    m = jnp.max(s, axis=1, keepdims=True)
    p = jnp.exp(s - m) * w_t
    l = jnp.sum(p, axis=1, keepdims=True)
    v16 = v_ref[:].astype(jnp.bfloat16)
    g = lax.dot_general(p.astype(jnp.bfloat16), v16,
                        (((1,), (0,)), ((), ())),
                        preferred_element_type=jnp.float32)
    for h in range(H):
        comm_out[:, h * D:(h + 1) * D] = g[h * B:(h + 1) * B, h * D:(h + 1) * D]
        comm_out[:, MCOL + h:MCOL + h + 1] = m[h * B:(h + 1) * B, :]
        comm_out[:, LCOL + h:LCOL + h + 1] = l[h * B:(h + 1) * B, :]

    rdma = pltpu.make_async_remote_copy(
        src_ref=comm_out, dst_ref=comm_in,
        send_sem=send_sem, recv_sem=recv_sem,
        device_id=peer, device_id_type=pl.DeviceIdType.MESH,
    )
    rdma.start()
    rdma.wait()

    m_l = comm_out[:, MCOL:MCOL + H]
    l_l = comm_out[:, LCOL:LCOL + H]
    m_r = comm_in[:, MCOL:MCOL + H]
    l_r = comm_in[:, LCOL:LCOL + H]
    m_c = jnp.maximum(m_l, m_r)
    a_l = jnp.exp(m_l - m_c)
    a_r = jnp.exp(m_r - m_c)
    l_c = a_l * l_l + a_r * l_r

    hr = lax.broadcasted_iota(jnp.int32, (H, HD), 0)
    hc = lax.broadcasted_iota(jnp.int32, (H, HD), 1)
    e2 = (hr == hc // D).astype(jnp.float32)

    def exp_cols(a):
        return lax.dot_general(a, e2, (((1,), (0,)), ((), ())),
                               preferred_element_type=jnp.float32)

    out_ref[:, :] = (exp_cols(a_l) * comm_out[:, :HD]
                     + exp_cols(a_r) * comm_in[:, :HD]) / exp_cols(l_c)


def kernel(Q, K, V, bt, lens):
    q2 = Q.reshape(B, HD)
    k2 = K.reshape(NKEYS, HD)
    v2 = V.reshape(NKEYS, HD)
    lens2 = lens.reshape(B, 1)
    out = pl.pallas_call(
        _body,
        out_shape=jax.ShapeDtypeStruct((B, HD), jnp.float32),
        in_specs=[pl.BlockSpec(memory_space=pltpu.VMEM)] * 5,
        out_specs=pl.BlockSpec(memory_space=pltpu.VMEM),
        scratch_shapes=[
            pltpu.VMEM((B, CW), jnp.float32),
            pltpu.VMEM((B, CW), jnp.float32),
            pltpu.SemaphoreType.DMA,
            pltpu.SemaphoreType.DMA,
        ],
        compiler_params=pltpu.CompilerParams(collective_id=0),
    )(q2, k2, v2, bt, lens2)
    return out.reshape(B, 1, H, D)


# device time: 18152 ns/iter; 3.4361x vs baseline; 3.1914x over previous
import jax
import jax.numpy as jnp
from jax import lax
from jax.experimental import pallas as pl
from jax.experimental.pallas import tpu as pltpu

B, H, D, BS = 16, 16, 64, 16
NB = 128
NPAGES_LOCAL = 128
HD = H * D
MCOL = HD
LCOL = HD + H
CW = HD + 2 * H

NEG = -1e30


def _body(q_ref, k_ref, v_ref, bt_ref, lens_ref, out_ref,
          kt, vt, comm_out, comm_in, k_sems, v_sems, send_sem, recv_sem):
    my_x = lax.axis_index("x")
    peer = (1 - my_x, lax.axis_index("y"), lax.axis_index("z"))

    k_copy = pltpu.make_async_copy(k_ref, kt, k_sems.at[0])
    v_copy = pltpu.make_async_copy(v_ref, vt, v_sems.at[0])
    k_copy.start()
    v_copy.start()

    barrier = pltpu.get_barrier_semaphore()
    pl.semaphore_signal(barrier, inc=1, device_id=peer,
                        device_id_type=pl.DeviceIdType.MESH)
    pl.semaphore_wait(barrier, 1)

    bt3 = bt_ref[:][:, :, None]
    pages = lax.broadcasted_iota(jnp.int32, (B, NB, NPAGES_LOCAL), 2) \
        + my_x * NPAGES_LOCAL
    slots = lax.broadcasted_iota(jnp.int32, (B, NB, NPAGES_LOCAL), 1)
    lens3 = lens_ref[:][:, :, None]
    hit = (bt3 == pages) & (slots < lens3)
    cnt = jnp.sum(hit.astype(jnp.float32), axis=1)

    logw = jnp.log(cnt)
    logw_t = jnp.broadcast_to(logw[None], (H, B, NPAGES_LOCAL)) \
        .reshape(H * B, NPAGES_LOCAL)

    scale = D ** -0.5
    q2 = q_ref[:].reshape(B, HD)
    q_t = jnp.broadcast_to(q2[None], (H, B, HD)).reshape(H * B, HD)
    rowh = lax.broadcasted_iota(jnp.int32, (H * B, HD), 0) // B
    colh = lax.broadcasted_iota(jnp.int32, (H * B, HD), 1) // D
    qbig = jnp.where(rowh == colh, q_t, 0.0)

    s_blks = []
    m = jnp.full((H * B, 1), NEG, jnp.float32)
    k_copy.wait()
    for blk in range(BS):
        kb = kt[blk].reshape(HD, NPAGES_LOCAL)
        s = lax.dot_general(qbig, kb, (((1,), (0,)), ((), ())),
                            preferred_element_type=jnp.float32) * scale
        s = s + logw_t
        s_blks.append(s)
        m = jnp.maximum(m, jnp.max(s, axis=1, keepdims=True))

    l = jnp.zeros((H * B, 1), jnp.float32)
    g = jnp.zeros((H * B, HD), jnp.float32)
    v_copy.wait()
    for blk in range(BS):
        p = jnp.exp(s_blks[blk] - m)
        l = l + jnp.sum(p, axis=1, keepdims=True)
        vb = vt[blk].reshape(HD, NPAGES_LOCAL)
        g = g + lax.dot_general(p, vb, (((1,), (1,)), ((), ())),
                                preferred_element_type=jnp.float32)

    for h in range(H):
        comm_out[:, h * D:(h + 1) * D] = g[h * B:(h + 1) * B, h * D:(h + 1) * D]
        comm_out[:, MCOL + h:MCOL + h + 1] = m[h * B:(h + 1) * B, :]
        comm_out[:, LCOL + h:LCOL + h + 1] = l[h * B:(h + 1) * B, :]

    rdma = pltpu.make_async_remote_copy(
        src_ref=comm_out, dst_ref=comm_in,
        send_sem=send_sem, recv_sem=recv_sem,
        device_id=peer, device_id_type=pl.DeviceIdType.MESH,
    )
    rdma.start()
    rdma.wait()

    m_l = comm_out[:, MCOL:MCOL + H]
    l_l = comm_out[:, LCOL:LCOL + H]
    m_r = comm_in[:, MCOL:MCOL + H]
    l_r = comm_in[:, LCOL:LCOL + H]
    m_c = jnp.maximum(m_l, m_r)
    a_l = jnp.exp(m_l - m_c)
    a_r = jnp.exp(m_r - m_c)
    l_c = a_l * l_l + a_r * l_r

    hr = lax.broadcasted_iota(jnp.int32, (H, HD), 0)
    hc = lax.broadcasted_iota(jnp.int32, (H, HD), 1)
    e2 = (hr == hc // D).astype(jnp.float32)

    def exp_cols(a):
        return lax.dot_general(a, e2, (((1,), (0,)), ((), ())),
                               preferred_element_type=jnp.float32)

    out2 = (exp_cols(a_l) * comm_out[:, :HD]
            + exp_cols(a_r) * comm_in[:, :HD]) / exp_cols(l_c)
    out_ref[:] = out2.reshape(B, 1, H, D)


def kernel(Q, K, V, bt, lens):
    KT = jnp.transpose(K, (1, 2, 3, 0))
    VT = jnp.transpose(V, (1, 2, 3, 0))
    return pl.pallas_call(
        _body,
        out_shape=jax.ShapeDtypeStruct((B, 1, H, D), jnp.float32),
        in_specs=[
            pl.BlockSpec(memory_space=pltpu.VMEM),
            pl.BlockSpec(memory_space=pl.ANY),
            pl.BlockSpec(memory_space=pl.ANY),
            pl.BlockSpec(memory_space=pltpu.VMEM),
            pl.BlockSpec(memory_space=pltpu.VMEM),
        ],
        out_specs=pl.BlockSpec(memory_space=pltpu.VMEM),
        scratch_shapes=[
            pltpu.VMEM((BS, H, D, NPAGES_LOCAL), jnp.float32),
            pltpu.VMEM((BS, H, D, NPAGES_LOCAL), jnp.float32),
            pltpu.VMEM((B, CW), jnp.float32),
            pltpu.VMEM((B, CW), jnp.float32),
            pltpu.SemaphoreType.DMA((BS,)),
            pltpu.SemaphoreType.DMA((BS,)),
            pltpu.SemaphoreType.DMA,
            pltpu.SemaphoreType.DMA,
        ],
        compiler_params=pltpu.CompilerParams(
            collective_id=0, vmem_limit_bytes=100 * 1024 * 1024),
    )(Q, KT, VT, bt, lens.reshape(B, 1))
